# baseline (speedup 1.0000x reference)
import functools

import jax
import jax.numpy as jnp
from jax import lax
from jax.experimental import pallas as pl
from jax.experimental.pallas import tpu as pltpu
from jax.experimental.pallas import tpu_sc as plsc

NC = 2
NS = 16
NW = NC * NS
CHUNK = 128
NBUF_X = 2
GPS = 2
NBUF_C = 2


def _make_super_ring(table_hbm, idx_v, out_hbm, out_base, buf_v, gsem, osem,
                     nbuf):
    SUP = GPS * CHUNK

    def start_gathers(s, b):
        for h in range(GPS):
            pltpu.make_async_copy(
                table_hbm.at[idx_v.at[pl.ds((s * GPS + h) * CHUNK, CHUNK)]],
                buf_v.at[b, pl.ds(h * CHUNK, CHUNK)], gsem.at[b]).start()

    def wait_gathers(b):
        pltpu.make_async_copy(
            table_hbm.at[idx_v.at[pl.ds(0, CHUNK)]],
            buf_v.at[b], gsem.at[b]).wait()

    def start_out(s, b):
        pltpu.make_async_copy(
            buf_v.at[b],
            out_hbm.at[pl.ds(out_base + s * SUP, SUP)],
            osem.at[b]).start()

    def wait_out(b):
        pltpu.make_async_copy(
            buf_v.at[b],
            out_hbm.at[pl.ds(out_base, SUP)],
            osem.at[b]).wait()

    def prime():
        for b in range(nbuf):
            start_gathers(b, b)

    def run(nsup):
        ngroups = nsup // nbuf

        def group_body(g, carry):
            base_s = g * nbuf
            for b in range(nbuf):
                wait_gathers(b)
                start_out(base_s + b, b)
                wait_out(b)
                start_gathers(base_s + b + nbuf, b)
            return carry

        if ngroups > 1:
            lax.fori_loop(0, ngroups - 1, group_body, 0)

        base_s = (ngroups - 1) * nbuf
        for b in range(nbuf):
            wait_gathers(b)
            start_out(base_s + b, b)
        for b in range(nbuf):
            wait_out(b)

    return prime, run


def _make_ring(table_hbm, idx_v, out_hbm, out_base, buf_v, gsem, osem, nbuf):
    def start_gather(c, b):
        pltpu.make_async_copy(
            table_hbm.at[idx_v.at[pl.ds(c * CHUNK, CHUNK)]],
            buf_v.at[b], gsem.at[b]).start()

    def wait_gather(b):
        pltpu.make_async_copy(
            table_hbm.at[idx_v.at[pl.ds(0, CHUNK)]],
            buf_v.at[b], gsem.at[b]).wait()

    def start_out(c, b):
        pltpu.make_async_copy(
            buf_v.at[b],
            out_hbm.at[pl.ds(out_base + c * CHUNK, CHUNK)],
            osem.at[b]).start()

    def wait_out(b):
        pltpu.make_async_copy(
            buf_v.at[b],
            out_hbm.at[pl.ds(out_base, CHUNK)],
            osem.at[b]).wait()

    def prime():
        for b in range(nbuf):
            start_gather(b, b)

    def run(nchunks):
        ngroups = nchunks // nbuf

        def group_body(g, carry):
            base_c = g * nbuf
            for b in range(nbuf):
                wait_gather(b)
                start_out(base_c + b, b)
                wait_out(b)
                start_gather(base_c + b + nbuf, b)
            return carry

        if ngroups > 1:
            lax.fori_loop(0, ngroups - 1, group_body, 0)

        base_c = (ngroups - 1) * nbuf
        for b in range(nbuf):
            wait_gather(b)
            start_out(base_c + b, b)
        for b in range(nbuf):
            wait_out(b)

    return prime, run


@functools.lru_cache(maxsize=None)
def _build(B, C, V, D):
    assert D % 16 == 0
    n_center = B // NW
    n_ctx = (B * C) // NW
    assert B % NW == 0 and (B * C) % NW == 0
    assert n_center % CHUNK == 0 and n_ctx % CHUNK == 0
    assert (n_center // CHUNK) % NBUF_C == 0
    assert (n_ctx // (GPS * CHUNK)) % NBUF_X == 0

    mesh = plsc.VectorSubcoreMesh(core_axis_name="c", subcore_axis_name="s")

    @functools.partial(
        pl.kernel,
        mesh=mesh,
        out_type=(
            jax.ShapeDtypeStruct((B, D), jnp.float32),
            jax.ShapeDtypeStruct((C * B, D), jnp.float32),
        ),
        scratch_types=[
            pltpu.VMEM((n_center,), jnp.int32),
            pltpu.VMEM((n_ctx,), jnp.int32),
            pltpu.VMEM((NBUF_C, CHUNK, D), jnp.float32),
            pltpu.VMEM((NBUF_X, GPS * CHUNK, D), jnp.float32),
            pltpu.SemaphoreType.DMA((NBUF_C,)),
            pltpu.SemaphoreType.DMA((NBUF_C,)),
            pltpu.SemaphoreType.DMA((NBUF_X,)),
            pltpu.SemaphoreType.DMA((NBUF_X,)),
        ],
    )
    def sc_kernel(center_hbm, ctx_hbm, table_hbm, outc_hbm, outx_hbm,
                  idxc_v, idxx_v, bufc_v, bufx_v, cgsem, cosem, xgsem, xosem):
        wid = lax.axis_index("s") * NC + lax.axis_index("c")
        pltpu.sync_copy(center_hbm.at[pl.ds(wid * n_center, n_center)], idxc_v)
        pltpu.sync_copy(ctx_hbm.at[pl.ds(wid * n_ctx, n_ctx)], idxx_v)
        c_prime, c_run = _make_ring(table_hbm, idxc_v, outc_hbm,
                                    wid * n_center, bufc_v, cgsem, cosem,
                                    NBUF_C)
        x_prime, x_run = _make_super_ring(table_hbm, idxx_v, outx_hbm,
                                          wid * n_ctx, bufx_v, xgsem, xosem,
                                          NBUF_X)
        c_prime()
        x_prime()
        c_run(n_center // CHUNK)
        x_run(n_ctx // (GPS * CHUNK))

    return sc_kernel


def kernel(center, context, embedding):
    B, C = context.shape
    V, D = embedding.shape
    sc_kernel = _build(B, C, V, D)
    outc, outx = sc_kernel(
        center.astype(jnp.int32),
        context.T.reshape(C * B).astype(jnp.int32),
        embedding,
    )
    return outc, outx.reshape(C, B, D).transpose(1, 0, 2)


# speedup vs baseline: 1.0060x; 1.0060x over previous
import functools

import jax
import jax.numpy as jnp
from jax import lax
from jax.experimental import pallas as pl
from jax.experimental.pallas import tpu as pltpu
from jax.experimental.pallas import tpu_sc as plsc

NC = 2
NS = 16
NW = NC * NS
CHUNK = 128
NBUF_X = 5
NBUF_C = 2


def _make_ring(table_hbm, idx_v, out_hbm, out_base, buf_v, gsem, osem, nbuf):
    def start_gather(c, b):
        pltpu.make_async_copy(
            table_hbm.at[idx_v.at[pl.ds(c * CHUNK, CHUNK)]],
            buf_v.at[b], gsem.at[b]).start()

    def wait_gather(b):
        pltpu.make_async_copy(
            table_hbm.at[idx_v.at[pl.ds(0, CHUNK)]],
            buf_v.at[b], gsem.at[b]).wait()

    def start_out(c, b):
        pltpu.make_async_copy(
            buf_v.at[b],
            out_hbm.at[pl.ds(out_base + c * CHUNK, CHUNK)],
            osem.at[b]).start()

    def wait_out(b):
        pltpu.make_async_copy(
            buf_v.at[b],
            out_hbm.at[pl.ds(out_base, CHUNK)],
            osem.at[b]).wait()

    def prime():
        for b in range(nbuf):
            start_gather(b, b)

    def run(nchunks):
        ngroups = nchunks // nbuf

        def group_body(g, carry):
            base_c = g * nbuf
            for b in range(nbuf):
                wait_gather(b)
                start_out(base_c + b, b)
                wait_out(b)
                start_gather(base_c + b + nbuf, b)
            return carry

        if ngroups > 1:
            lax.fori_loop(0, ngroups - 1, group_body, 0)

        base_c = (ngroups - 1) * nbuf
        for b in range(nbuf):
            wait_gather(b)
            start_out(base_c + b, b)
        for b in range(nbuf):
            wait_out(b)

    return prime, run


@functools.lru_cache(maxsize=None)
def _build(B, C, V, D):
    assert D % 16 == 0
    n_center = B // NW
    n_ctx = (B * C) // NW
    assert B % NW == 0 and (B * C) % NW == 0
    assert n_center % CHUNK == 0 and n_ctx % CHUNK == 0
    assert (n_center // CHUNK) % NBUF_C == 0
    assert (n_ctx // CHUNK) % NBUF_X == 0

    mesh = plsc.VectorSubcoreMesh(core_axis_name="c", subcore_axis_name="s")

    @functools.partial(
        pl.kernel,
        mesh=mesh,
        out_type=(
            jax.ShapeDtypeStruct((B, D), jnp.float32),
            jax.ShapeDtypeStruct((C * B, D), jnp.float32),
        ),
        scratch_types=[
            pltpu.VMEM((n_center,), jnp.int32),
            pltpu.VMEM((n_ctx,), jnp.int32),
            pltpu.VMEM((NBUF_C, CHUNK, D), jnp.float32),
            pltpu.VMEM((NBUF_X, CHUNK, D), jnp.float32),
            pltpu.SemaphoreType.DMA((NBUF_C,)),
            pltpu.SemaphoreType.DMA((NBUF_C,)),
            pltpu.SemaphoreType.DMA((NBUF_X,)),
            pltpu.SemaphoreType.DMA((NBUF_X,)),
        ],
    )
    def sc_kernel(center_hbm, ctx_hbm, table_hbm, outc_hbm, outx_hbm,
                  idxc_v, idxx_v, bufc_v, bufx_v, cgsem, cosem, xgsem, xosem):
        wid = lax.axis_index("s") * NC + lax.axis_index("c")
        pltpu.sync_copy(center_hbm.at[pl.ds(wid * n_center, n_center)], idxc_v)
        pltpu.sync_copy(ctx_hbm.at[pl.ds(wid * n_ctx, n_ctx)], idxx_v)
        c_prime, c_run = _make_ring(table_hbm, idxc_v, outc_hbm,
                                    wid * n_center, bufc_v, cgsem, cosem,
                                    NBUF_C)
        x_prime, x_run = _make_ring(table_hbm, idxx_v, outx_hbm,
                                    wid * n_ctx, bufx_v, xgsem, xosem,
                                    NBUF_X)
        c_prime()
        x_prime()
        c_run(n_center // CHUNK)
        x_run(n_ctx // CHUNK)

    return sc_kernel


def kernel(center, context, embedding):
    B, C = context.shape
    V, D = embedding.shape
    sc_kernel = _build(B, C, V, D)
    outc, outx = sc_kernel(
        center.astype(jnp.int32),
        context.T.reshape(C * B).astype(jnp.int32),
        embedding,
    )
    return outc, outx.reshape(C, B, D).transpose(1, 0, 2)
